# baseline (speedup 1.0000x reference)
import jax
import jax.numpy as jnp
from jax.experimental import pallas as pl

_BQ = 2048
_NPAD = 2048


def _count_block(n_ref, cali_ref, out_ref, *, inv_n):
    s = jnp.sum(n_ref[...], axis=1, keepdims=True)
    ge = (cali_ref[...] >= s).astype(jnp.float32)
    out_ref[...] = jnp.sum(ge, axis=1, keepdims=True) * inv_n


def kernel(nonconformity, label_sample, cali_nonconformity):
    B, J, C = nonconformity.shape
    N = cali_nonconformity.shape[0]
    Q = B * C
    nq = nonconformity.transpose(0, 2, 1).reshape(Q, J)
    cali_pad = jnp.concatenate(
        [cali_nonconformity,
         jnp.full((_NPAD - N,), -jnp.inf, jnp.float32)]).reshape(1, _NPAD)

    import functools
    body = functools.partial(_count_block, inv_n=1.0 / N)
    p = pl.pallas_call(
        body,
        grid=(Q // _BQ,),
        in_specs=[
            pl.BlockSpec((_BQ, J), lambda i: (i, 0)),
            pl.BlockSpec((1, _NPAD), lambda i: (0, 0)),
        ],
        out_specs=pl.BlockSpec((_BQ, 1), lambda i: (i, 0)),
        out_shape=jax.ShapeDtypeStruct((Q, 1), jnp.float32),
    )(nq, cali_pad)
    return p.reshape(B, C)


# speedup vs baseline: 1.2617x; 1.2617x over previous
import functools

import jax
import jax.numpy as jnp
from jax import lax
from jax.experimental import pallas as pl
from jax.experimental.pallas import tpu as pltpu
from jax.experimental.pallas import tpu_sc as plsc

_NPAD = 2048
_SORT_BLK = 512
_NC, _NS, _L = 2, 16, 16
_NW = _NC * _NS


def _sort_body(c_col_ref, c_row_ref, out_ref):
    g = pl.program_id(0)
    R = _SORT_BLK
    ccol = c_col_ref[...]
    crow = c_row_ref[...]
    ic = lax.broadcasted_iota(jnp.int32, (R, _NPAD), 0) + g * R
    ir = lax.broadcasted_iota(jnp.int32, (R, _NPAD), 1)
    lt = (crow < ccol) | ((crow == ccol) & (ir < ic))
    rank = jnp.sum(lt.astype(jnp.float32), axis=1,
                   keepdims=True).astype(jnp.int32)
    kr = lax.broadcasted_iota(jnp.int32, (R, _NPAD), 1)
    contrib = jnp.sum(jnp.where(rank == kr, ccol, 0.0), axis=0,
                      keepdims=True)

    @pl.when(g == 0)
    def _():
        out_ref[...] = contrib

    @pl.when(g > 0)
    def _():
        out_ref[...] += contrib


def _sort_tc(cali_pad):
    c_col = cali_pad.reshape(_NPAD, 1)
    c_row = cali_pad.reshape(1, _NPAD)
    out = pl.pallas_call(
        _sort_body,
        grid=(_NPAD // _SORT_BLK,),
        in_specs=[
            pl.BlockSpec((_SORT_BLK, 1), lambda i: (i, 0)),
            pl.BlockSpec((1, _NPAD), lambda i: (0, 0)),
        ],
        out_specs=pl.BlockSpec((1, _NPAD), lambda i: (0, 0)),
        out_shape=jax.ShapeDtypeStruct((1, _NPAD), jnp.float32),
    )(c_col, c_row)
    return out.reshape(_NPAD)


def _make_sc_search(B, J, C, N, Q):
    QW = Q // _NW
    ZW = B // _NW
    CHUNK = ZW * J * C

    DIV_SHIFT = 16
    while True:
        DIV_MULT = -(-(1 << DIV_SHIFT) // C)
        if all((k * DIV_MULT) >> DIV_SHIFT == k // C for k in range(QW)):
            break
        DIV_SHIFT += 1
    mesh = plsc.VectorSubcoreMesh(core_axis_name="c", subcore_axis_name="s")

    @functools.partial(
        pl.kernel,
        out_type=jax.ShapeDtypeStruct((Q,), jnp.float32),
        mesh=mesh,
        compiler_params=pltpu.CompilerParams(needs_layout_passes=False),
        scratch_types=[
            pltpu.VMEM((CHUNK,), jnp.float32),
            pltpu.VMEM((_NPAD,), jnp.float32),
            pltpu.VMEM((QW,), jnp.float32),
        ],
    )
    def sc_search(n_hbm, sorted_hbm, out_hbm, n_v, cali_v, out_v):
        wid = lax.axis_index("s") * _NC + lax.axis_index("c")
        pltpu.sync_copy(n_hbm.at[pl.ds(wid * CHUNK, CHUNK)], n_v)
        pltpu.sync_copy(sorted_hbm, cali_v)
        lanes = lax.iota(jnp.int32, _L)

        def body(v, carry):
            ql = v * _L + lanes
            z = (ql * DIV_MULT) >> DIV_SHIFT
            i = ql - z * C
            base = z * (J * C) + i
            s = plsc.load_gather(n_v, [base])
            for j in range(1, J):
                s = s + plsc.load_gather(n_v, [base + j * C])
            lo = jnp.zeros((_L,), jnp.int32)
            hi = jnp.full((_L,), _NPAD, jnp.int32)
            for _ in range(11):
                mid = (lo + hi) >> 1
                vv = plsc.load_gather(cali_v, [mid])
                pred = vv < s
                lo = jnp.where(pred, mid + 1, lo)
                hi = jnp.where(pred, hi, mid)
            p = (jnp.float32(N) - lo.astype(jnp.float32)) / jnp.float32(N)
            out_v[pl.ds(v * _L, _L)] = p
            return carry

        lax.fori_loop(0, QW // _L, body, 0)
        pltpu.sync_copy(out_v, out_hbm.at[pl.ds(wid * QW, QW)])

    return sc_search


def kernel(nonconformity, label_sample, cali_nonconformity):
    B, J, C = nonconformity.shape
    N = cali_nonconformity.shape[0]
    Q = B * C
    cali_pad = jnp.concatenate(
        [cali_nonconformity,
         jnp.full((_NPAD - N,), jnp.inf, jnp.float32)])
    sorted_cali = _sort_tc(cali_pad)
    n_flat = nonconformity.reshape(B * J * C)
    p_flat = _make_sc_search(B, J, C, N, Q)(n_flat, sorted_cali)
    return p_flat.reshape(B, C)


# speedup vs baseline: 1.4086x; 1.1164x over previous
import functools

import jax
import jax.numpy as jnp
from jax import lax
from jax.experimental import pallas as pl
from jax.experimental.pallas import tpu as pltpu
from jax.experimental.pallas import tpu_sc as plsc

_NPAD = 2048
_SORT_BLK = 512
_NC, _NS, _L = 2, 16, 16
_NW = _NC * _NS


def _sort_body(c_col_ref, c_row_ref, out_ref):
    g = pl.program_id(0)
    R = _SORT_BLK
    ccol = c_col_ref[...]
    crow = c_row_ref[...]
    ic = lax.broadcasted_iota(jnp.int32, (R, _NPAD), 0) + g * R
    ir = lax.broadcasted_iota(jnp.int32, (R, _NPAD), 1)
    lt = (crow < ccol) | ((crow == ccol) & (ir < ic))
    rank = jnp.sum(lt.astype(jnp.float32), axis=1,
                   keepdims=True).astype(jnp.int32)
    kr = lax.broadcasted_iota(jnp.int32, (R, _NPAD), 1)
    contrib = jnp.sum(jnp.where(rank == kr, ccol, 0.0), axis=0,
                      keepdims=True)

    @pl.when(g == 0)
    def _():
        out_ref[...] = contrib

    @pl.when(g > 0)
    def _():
        out_ref[...] += contrib


def _sort_tc(cali_pad):
    c_col = cali_pad.reshape(_NPAD, 1)
    c_row = cali_pad.reshape(1, _NPAD)
    out = pl.pallas_call(
        _sort_body,
        grid=(_NPAD // _SORT_BLK,),
        in_specs=[
            pl.BlockSpec((_SORT_BLK, 1), lambda i: (i, 0)),
            pl.BlockSpec((1, _NPAD), lambda i: (0, 0)),
        ],
        out_specs=pl.BlockSpec((1, _NPAD), lambda i: (0, 0)),
        out_shape=jax.ShapeDtypeStruct((1, _NPAD), jnp.float32),
    )(c_col, c_row)
    return out.reshape(_NPAD)


def _make_sc_search(B, J, C, N, Q):
    QW = Q // _NW
    ZW = B // _NW
    CHUNK = ZW * J * C

    DIV_SHIFT = 16
    while True:
        DIV_MULT = -(-(1 << DIV_SHIFT) // C)
        if all((k * DIV_MULT) >> DIV_SHIFT == k // C for k in range(QW)):
            break
        DIV_SHIFT += 1
    mesh = plsc.VectorSubcoreMesh(core_axis_name="c", subcore_axis_name="s")

    @functools.partial(
        pl.kernel,
        out_type=jax.ShapeDtypeStruct((Q,), jnp.float32),
        mesh=mesh,
        compiler_params=pltpu.CompilerParams(needs_layout_passes=False),
        scratch_types=[
            pltpu.VMEM((CHUNK,), jnp.float32),
            pltpu.VMEM((_NPAD,), jnp.float32),
            pltpu.VMEM((QW,), jnp.float32),
        ],
    )
    def sc_search(n_hbm, sorted_hbm, out_hbm, n_v, cali_v, out_v):
        wid = lax.axis_index("s") * _NC + lax.axis_index("c")
        pltpu.sync_copy(n_hbm.at[pl.ds(wid * CHUNK, CHUNK)], n_v)
        pltpu.sync_copy(sorted_hbm, cali_v)
        lanes = lax.iota(jnp.int32, _L)

        @plsc.parallel_loop(0, QW // _L, step=1, unroll=4)
        def body(v):
            ql = v * _L + lanes
            z = (ql * DIV_MULT) >> DIV_SHIFT
            i = ql - z * C
            base = z * (J * C) + i
            s = plsc.load_gather(n_v, [base])
            for j in range(1, J):
                s = s + plsc.load_gather(n_v, [base + j * C])
            lo = jnp.zeros((_L,), jnp.int32)
            hi = jnp.full((_L,), _NPAD, jnp.int32)
            for _ in range(11):
                mid = (lo + hi) >> 1
                vv = plsc.load_gather(cali_v, [mid])
                pred = vv < s
                lo = jnp.where(pred, mid + 1, lo)
                hi = jnp.where(pred, hi, mid)
            p = (jnp.float32(N) - lo.astype(jnp.float32)) / jnp.float32(N)
            out_v[pl.ds(v * _L, _L)] = p
        pltpu.sync_copy(out_v, out_hbm.at[pl.ds(wid * QW, QW)])

    return sc_search


def kernel(nonconformity, label_sample, cali_nonconformity):
    B, J, C = nonconformity.shape
    N = cali_nonconformity.shape[0]
    Q = B * C
    cali_pad = jnp.concatenate(
        [cali_nonconformity,
         jnp.full((_NPAD - N,), jnp.inf, jnp.float32)])
    sorted_cali = _sort_tc(cali_pad)
    n_flat = nonconformity.reshape(B * J * C)
    p_flat = _make_sc_search(B, J, C, N, Q)(n_flat, sorted_cali)
    return p_flat.reshape(B, C)
